# baseline (speedup 1.0000x reference)
import functools

import jax
import jax.numpy as jnp
from jax import lax
from jax.experimental import pallas as pl
from jax.experimental.pallas import tpu as pltpu
from jax.experimental.pallas import tpu_sc as plsc


def _lookup_kernel(B, D, V, NW, b_per_w, C):
    mesh = plsc.VectorSubcoreMesh(core_axis_name="c", subcore_axis_name="s")
    n_chunks = b_per_w // C
    n_pieces = D // 16
    GR = 16

    @functools.partial(
        pl.kernel,
        mesh=mesh,
        out_type=jax.ShapeDtypeStruct((B, D), jnp.float32),
        scratch_types=[
            pltpu.VMEM((b_per_w,), jnp.int32),
            pltpu.VMEM((V, D), jnp.float32),
            pltpu.VMEM((2, C, D), jnp.float32),
            pltpu.SemaphoreType.DMA,
            pltpu.SemaphoreType.DMA,
        ],
    )
    def k(x_hbm, table_hbm, out_hbm, idx_v, table_v, rows_v, s0, s1):
        wid = lax.axis_index("s") * 2 + lax.axis_index("c")
        base = wid * b_per_w
        pltpu.sync_copy(x_hbm.at[pl.ds(base, b_per_w)], idx_v)
        pltpu.sync_copy(table_hbm, table_v)

        ssems = (s0, s1)

        def store(c, b):
            return pltpu.make_async_copy(
                rows_v.at[b], out_hbm.at[pl.ds(base + c * C, C)], ssems[b]
            )

        def assemble(c, b):
            @plsc.parallel_loop(0, C // GR, 1, unroll=1)
            def grp(g):
                row0 = g * GR
                iv = idx_v[pl.ds(c * C + row0, GR)]
                for r in range(GR):
                    xi = iv[r]
                    for p in range(n_pieces):
                        rows_v[b, row0 + r, pl.ds(p * 16, 16)] = table_v[
                            xi, pl.ds(p * 16, 16)
                        ]

        assemble(0, 0)
        store(0, 0).start()

        def body(c, _):
            for b in range(2):
                cc = c + b
                nxt = 1 - b
                @pl.when(cc >= 1)
                def _():
                    store(cc - 1, nxt).wait()

                @pl.when(cc + 1 < n_chunks)
                def _():
                    assemble(cc + 1, nxt)
                    store(cc + 1, nxt).start()

            return _

        lax.fori_loop(0, n_chunks // 2, lambda c, u: body(c * 2, u), None)
        store(n_chunks - 1, (n_chunks - 1) % 2).wait()

    return k


def kernel(x, table):
    S, J = x.shape
    V, D = table.shape
    B = S * J
    NW = 32
    b_per_w = B // NW
    C = 64
    xf = x.reshape(B).astype(jnp.int32)
    out = _lookup_kernel(B, D, V, NW, b_per_w, C)(xf, table)
    return out.reshape(S, J, D)


# speedup vs baseline: 2.6007x; 2.6007x over previous
import functools

import jax
import jax.numpy as jnp
from jax import lax
from jax.experimental import pallas as pl
from jax.experimental.pallas import tpu as pltpu
from jax.experimental.pallas import tpu_sc as plsc


def _gather_kernel(B, D, V, NW, b_per_w, C):
    mesh = plsc.VectorSubcoreMesh(core_axis_name="c", subcore_axis_name="s")
    n_chunks = b_per_w // C

    @functools.partial(
        pl.kernel,
        mesh=mesh,
        out_type=jax.ShapeDtypeStruct((B, D), jnp.float32),
        scratch_types=[
            pltpu.VMEM((b_per_w,), jnp.int32),
            pltpu.VMEM((2, C, D), jnp.float32),
            pltpu.SemaphoreType.DMA,
            pltpu.SemaphoreType.DMA,
            pltpu.SemaphoreType.DMA,
            pltpu.SemaphoreType.DMA,
        ],
    )
    def k(x_hbm, table_hbm, out_hbm, idx_v, rows_v, g0, g1, s0, s1):
        wid = lax.axis_index("s") * 2 + lax.axis_index("c")
        base = wid * b_per_w
        pltpu.sync_copy(x_hbm.at[pl.ds(base, b_per_w)], idx_v)

        gsems = (g0, g1)
        ssems = (s0, s1)

        def gather(c, b):
            return pltpu.make_async_copy(
                table_hbm.at[idx_v.at[pl.ds(c * C, C)]], rows_v.at[b], gsems[b]
            )

        def store(c, b):
            return pltpu.make_async_copy(
                rows_v.at[b], out_hbm.at[pl.ds(base + c * C, C)], ssems[b]
            )

        gather(0, 0).start()

        def body(c, _):
            for b in range(2):
                cc = c + b
                gather(cc, b).wait()
                store(cc, b).start()
                @pl.when(cc >= 1)
                def _():
                    store(cc - 1, 1 - b).wait()

                @pl.when(cc + 1 < n_chunks)
                def _():
                    gather(cc + 1, 1 - b).start()

            return _

        lax.fori_loop(0, n_chunks // 2, lambda c, u: body(c * 2, u), None)
        store(n_chunks - 1, (n_chunks - 1) % 2).wait()

    return k


def kernel(x, table):
    S, J = x.shape
    V, D = table.shape
    B = S * J
    NW = 32
    b_per_w = B // NW
    C = 64
    table_rep = jnp.broadcast_to(table[:, None, :], (V, NW, D)).reshape(
        V * NW, D
    )
    xf = x.reshape(B).astype(jnp.int32)
    xf = xf * NW + (jnp.arange(B, dtype=jnp.int32) // b_per_w)
    out = _gather_kernel(B, D, V, NW, b_per_w, C)(xf, table_rep)
    return out.reshape(S, J, D)


# speedup vs baseline: 2.6304x; 1.0114x over previous
import functools

import jax
import jax.numpy as jnp
from jax import lax
from jax.experimental import pallas as pl
from jax.experimental.pallas import tpu as pltpu
from jax.experimental.pallas import tpu_sc as plsc


def _gather_kernel(B, D, V, NW, b_per_w, C):
    mesh = plsc.VectorSubcoreMesh(core_axis_name="c", subcore_axis_name="s")
    n_chunks = b_per_w // C

    @functools.partial(
        pl.kernel,
        mesh=mesh,
        out_type=jax.ShapeDtypeStruct((B, D), jnp.float32),
        scratch_types=[
            pltpu.VMEM((b_per_w,), jnp.int32),
            pltpu.VMEM((2, C, D), jnp.float32),
            pltpu.SemaphoreType.DMA,
            pltpu.SemaphoreType.DMA,
            pltpu.SemaphoreType.DMA,
            pltpu.SemaphoreType.DMA,
        ],
    )
    def k(x_hbm, table_hbm, out_hbm, idx_v, rows_v, g0, g1, s0, s1):
        wid = lax.axis_index("s") * 2 + lax.axis_index("c")
        base = wid * b_per_w
        pltpu.sync_copy(x_hbm.at[pl.ds(base, b_per_w)], idx_v)

        gsems = (g0, g1)
        ssems = (s0, s1)

        def gather(c, b):
            return pltpu.make_async_copy(
                table_hbm.at[idx_v.at[pl.ds(c * C, C)]], rows_v.at[b], gsems[b]
            )

        def store(c, b):
            return pltpu.make_async_copy(
                rows_v.at[b], out_hbm.at[pl.ds(base + c * C, C)], ssems[b]
            )

        gather(0, 0).start()

        def body(c, _):
            for b in range(2):
                cc = c + b
                gather(cc, b).wait()
                store(cc, b).start()
                @pl.when(cc >= 1)
                def _():
                    store(cc - 1, 1 - b).wait()

                @pl.when(cc + 1 < n_chunks)
                def _():
                    gather(cc + 1, 1 - b).start()

            return _

        lax.fori_loop(0, n_chunks // 2, lambda c, u: body(c * 2, u), None)
        store(n_chunks - 1, (n_chunks - 1) % 2).wait()

    return k


def kernel(x, table):
    S, J = x.shape
    V, D = table.shape
    B = S * J
    NW = 32
    b_per_w = B // NW
    C = 32
    table_rep = jnp.tile(table, (NW, 1))
    xf = x.reshape(B).astype(jnp.int32)
    xf = xf + V * (jnp.arange(B, dtype=jnp.int32) // b_per_w)
    out = _gather_kernel(B, D, V, NW, b_per_w, C)(xf, table_rep)
    return out.reshape(S, J, D)


# speedup vs baseline: 2.8040x; 1.0660x over previous
import functools

import jax
import jax.numpy as jnp
from jax import lax
from jax.experimental import pallas as pl
from jax.experimental.pallas import tpu as pltpu
from jax.experimental.pallas import tpu_sc as plsc


def _lookup_kernel(B, D, V, NW, b_per_w, C, NG):
    mesh = plsc.VectorSubcoreMesh(core_axis_name="c", subcore_axis_name="s")
    n_chunks = b_per_w // C
    NA = n_chunks - NG
    n_groups = NA * 2
    n_pieces = D // 16
    GR = 16

    @functools.partial(
        pl.kernel,
        mesh=mesh,
        out_type=jax.ShapeDtypeStruct((B, D), jnp.float32),
        scratch_types=[
            pltpu.VMEM((b_per_w,), jnp.int32),
            pltpu.VMEM((V, D), jnp.float32),
            pltpu.VMEM((2, C, D), jnp.float32),
            pltpu.VMEM((2, C, D), jnp.float32),
            pltpu.SemaphoreType.DMA,
            pltpu.SemaphoreType.DMA,
            pltpu.SemaphoreType.DMA,
            pltpu.SemaphoreType.DMA,
            pltpu.SemaphoreType.DMA,
            pltpu.SemaphoreType.DMA,
        ],
    )
    def k(
        x_hbm,
        table_rep_hbm,
        table_hbm,
        out_hbm,
        idx_v,
        table_v,
        gbuf,
        abuf,
        g0,
        g1,
        s0,
        s1,
        a0,
        a1,
    ):
        wid = lax.axis_index("s") * 2 + lax.axis_index("c")
        base = wid * b_per_w
        pltpu.sync_copy(x_hbm.at[pl.ds(base, b_per_w)], idx_v)
        pltpu.sync_copy(table_hbm, table_v)

        gsems = (g0, g1)
        ssems = (s0, s1)
        asems = (a0, a1)

        def gather(c, b):
            return pltpu.make_async_copy(
                table_rep_hbm.at[idx_v.at[pl.ds(c * C, C)]],
                gbuf.at[b],
                gsems[b],
            )

        def store(c, b):
            return pltpu.make_async_copy(
                gbuf.at[b], out_hbm.at[pl.ds(base + c * C, C)], ssems[b]
            )

        def astore(c, sl, sem):
            return pltpu.make_async_copy(
                abuf.at[sl], out_hbm.at[pl.ds(base + c * C, C)], sem
            )

        def assemble_group(j):
            k_ = NG + j // 2
            row0 = (j % 2) * GR
            sl = (j // 2) % 2
            iv = idx_v[pl.ds(k_ * C + row0, GR)] - wid * V
            for r in range(GR):
                xi = iv[r]
                for p in range(n_pieces):
                    abuf[sl, row0 + r, pl.ds(p * 16, 16)] = table_v[
                        xi, pl.ds(p * 16, 16)
                    ]

        gather(0, 0).start()

        def body(c, _):
            for b in range(2):
                cc = c + b
                gather(cc, b).wait()
                store(cc, b).start()
                @pl.when(cc >= 1)
                def _():
                    store(cc - 1, 1 - b).wait()

                @pl.when(cc + 1 < NG)
                def _():
                    gather(cc + 1, 1 - b).start()

                @pl.when(cc < n_groups)
                def _():
                    j = cc
                    even = j % 2 == 0
                    parity0 = (j // 2) % 2 == 0

                    @pl.when(jnp.logical_and(even, j // 2 >= 2))
                    def _():
                        kprev = NG + j // 2 - 2

                        @pl.when(parity0)
                        def _():
                            astore(kprev, 0, a0).wait()

                        @pl.when(jnp.logical_not(parity0))
                        def _():
                            astore(kprev, 1, a1).wait()

                    assemble_group(j)

                    @pl.when(jnp.logical_not(even))
                    def _():
                        kcur = NG + j // 2

                        @pl.when(parity0)
                        def _():
                            astore(kcur, 0, a0).start()

                        @pl.when(jnp.logical_not(parity0))
                        def _():
                            astore(kcur, 1, a1).start()

            return _

        lax.fori_loop(0, NG // 2, lambda c, u: body(c * 2, u), None)
        store(NG - 1, (NG - 1) % 2).wait()
        astore(n_chunks - 2, (NA - 2) % 2, asems[(NA - 2) % 2]).wait()
        astore(n_chunks - 1, (NA - 1) % 2, asems[(NA - 1) % 2]).wait()

    return k


def kernel(x, table):
    S, J = x.shape
    V, D = table.shape
    B = S * J
    NW = 32
    b_per_w = B // NW
    C = 32
    NG = 44
    table_rep = jnp.tile(table, (NW, 1))
    xf = x.reshape(B).astype(jnp.int32)
    xr = xf + V * (jnp.arange(B, dtype=jnp.int32) // b_per_w)
    out = _lookup_kernel(B, D, V, NW, b_per_w, C, NG)(xr, table_rep, table)
    return out.reshape(S, J, D)


# speedup vs baseline: 2.8071x; 1.0011x over previous
import functools

import jax
import jax.numpy as jnp
from jax import lax
from jax.experimental import pallas as pl
from jax.experimental.pallas import tpu as pltpu
from jax.experimental.pallas import tpu_sc as plsc


def _lookup_kernel(B, D, V, NW, b_per_w, C, NG):
    mesh = plsc.VectorSubcoreMesh(core_axis_name="c", subcore_axis_name="s")
    n_chunks = b_per_w // C
    NA = n_chunks - NG
    n_groups = NA * 2
    n_pieces = D // 16
    GR = 16

    @functools.partial(
        pl.kernel,
        mesh=mesh,
        out_type=jax.ShapeDtypeStruct((B, D), jnp.float32),
        scratch_types=[
            pltpu.VMEM((b_per_w,), jnp.int32),
            pltpu.VMEM((V, D), jnp.float32),
            pltpu.VMEM((2, C, D), jnp.float32),
            pltpu.VMEM((2, C, D), jnp.float32),
            pltpu.SemaphoreType.DMA,
            pltpu.SemaphoreType.DMA,
            pltpu.SemaphoreType.DMA,
            pltpu.SemaphoreType.DMA,
            pltpu.SemaphoreType.DMA,
            pltpu.SemaphoreType.DMA,
        ],
    )
    def k(
        x_hbm,
        table_rep_hbm,
        table_hbm,
        out_hbm,
        idx_v,
        table_v,
        gbuf,
        abuf,
        g0,
        g1,
        s0,
        s1,
        a0,
        a1,
    ):
        wid = lax.axis_index("s") * 2 + lax.axis_index("c")
        base = wid * b_per_w
        pltpu.sync_copy(x_hbm.at[pl.ds(base, b_per_w)], idx_v)
        pltpu.sync_copy(table_hbm, table_v)

        gsems = (g0, g1)
        ssems = (s0, s1)
        asems = (a0, a1)

        def gather(c, b):
            return pltpu.make_async_copy(
                table_rep_hbm.at[idx_v.at[pl.ds(c * C, C)]],
                gbuf.at[b],
                gsems[b],
            )

        def store(c, b):
            return pltpu.make_async_copy(
                gbuf.at[b], out_hbm.at[pl.ds(base + c * C, C)], ssems[b]
            )

        def astore(c, sl, sem):
            return pltpu.make_async_copy(
                abuf.at[sl], out_hbm.at[pl.ds(base + c * C, C)], sem
            )

        def assemble_group(j):
            k_ = NG + j // 2
            row0 = (j % 2) * GR
            sl = (j // 2) % 2
            iv = idx_v[pl.ds(k_ * C + row0, GR)] - wid * V
            for r in range(GR):
                xi = iv[r]
                for p in range(n_pieces):
                    abuf[sl, row0 + r, pl.ds(p * 16, 16)] = table_v[
                        xi, pl.ds(p * 16, 16)
                    ]

        gather(0, 0).start()

        def body(c, _):
            for b in range(2):
                cc = c + b
                gather(cc, b).wait()
                store(cc, b).start()
                @pl.when(cc >= 1)
                def _():
                    store(cc - 1, 1 - b).wait()

                @pl.when(cc + 1 < NG)
                def _():
                    gather(cc + 1, 1 - b).start()

                @pl.when(cc < n_groups)
                def _():
                    j = cc
                    even = j % 2 == 0
                    parity0 = (j // 2) % 2 == 0

                    @pl.when(jnp.logical_and(even, j // 2 >= 2))
                    def _():
                        kprev = NG + j // 2 - 2

                        @pl.when(parity0)
                        def _():
                            astore(kprev, 0, a0).wait()

                        @pl.when(jnp.logical_not(parity0))
                        def _():
                            astore(kprev, 1, a1).wait()

                    assemble_group(j)

                    @pl.when(jnp.logical_not(even))
                    def _():
                        kcur = NG + j // 2

                        @pl.when(parity0)
                        def _():
                            astore(kcur, 0, a0).start()

                        @pl.when(jnp.logical_not(parity0))
                        def _():
                            astore(kcur, 1, a1).start()

            return _

        lax.fori_loop(0, NG // 2, lambda c, u: body(c * 2, u), None)
        store(NG - 1, (NG - 1) % 2).wait()
        astore(n_chunks - 2, (NA - 2) % 2, asems[(NA - 2) % 2]).wait()
        astore(n_chunks - 1, (NA - 1) % 2, asems[(NA - 1) % 2]).wait()

    return k


def kernel(x, table):
    S, J = x.shape
    V, D = table.shape
    B = S * J
    NW = 32
    b_per_w = B // NW
    C = 32
    NG = 46
    table_rep = jnp.tile(table, (NW, 1))
    xf = x.reshape(B).astype(jnp.int32)
    xr = xf + V * (jnp.arange(B, dtype=jnp.int32) // b_per_w)
    out = _lookup_kernel(B, D, V, NW, b_per_w, C, NG)(xr, table_rep, table)
    return out.reshape(S, J, D)
